# Initial kernel scaffold; baseline (speedup 1.0000x reference)
import jax
import jax.numpy as jnp
from jax.experimental import pallas as pl


def kernel(tc, embedding):
    raise NotImplementedError("write your pallas kernel here")


# speedup vs baseline: 3.6340x; 3.6340x over previous
import functools

import jax
import jax.numpy as jnp
from jax import lax
from jax.experimental import pallas as pl
from jax.experimental.pallas import tpu as pltpu
from jax.experimental.pallas import tpu_sc as plsc

_IDX_BLK = 128
_G = 8


@functools.lru_cache(maxsize=None)
def _build_gather(n_idx: int, n_rows: int, d: int):
    nc, ns = 2, 16
    nw = nc * ns
    rows_per_w = n_idx // (nw * _IDX_BLK)
    assert rows_per_w * nw * _IDX_BLK == n_idx
    steps = rows_per_w // _G
    assert steps * _G == rows_per_w
    chunk = _G * _IDX_BLK

    mesh = plsc.VectorSubcoreMesh(core_axis_name="c", subcore_axis_name="s")

    @functools.partial(
        pl.kernel,
        mesh=mesh,
        out_type=jax.ShapeDtypeStruct((n_idx, d), jnp.float32),
        scratch_types=[
            pltpu.VMEM((_G, _IDX_BLK), jnp.int32),
            pltpu.VMEM((chunk, d), jnp.float32),
            pltpu.SemaphoreType.DMA,
        ],
        compiler_params=pltpu.CompilerParams(use_tc_tiling_on_sc=False),
    )
    def gk(idx_hbm, table_hbm, out_hbm, idx_v, rows_v, sem):
        wid = lax.axis_index("s") * nc + lax.axis_index("c")
        base_row = wid * rows_per_w

        def body(ci, carry):
            r0 = base_row + ci * _G
            pltpu.sync_copy(idx_hbm.at[pl.ds(r0, _G)], idx_v)
            for j in range(_G):
                pltpu.async_copy(
                    table_hbm.at[idx_v.at[j]],
                    rows_v.at[pl.ds(j * _IDX_BLK, _IDX_BLK)],
                    sem,
                )
            for j in range(_G):
                pltpu.make_async_copy(
                    table_hbm.at[idx_v.at[j]],
                    rows_v.at[pl.ds(j * _IDX_BLK, _IDX_BLK)],
                    sem,
                ).wait()
            pltpu.sync_copy(rows_v, out_hbm.at[pl.ds(r0 * _IDX_BLK, chunk)])
            return carry

        lax.fori_loop(0, steps, body, 0, unroll=False)

    return gk


def kernel(tc, embedding):
    b, h = tc.shape
    v, d = embedding.shape
    n = b * h
    idx = tc.reshape(n // _IDX_BLK, _IDX_BLK).astype(jnp.int32)
    gk = _build_gather(n, v, d)
    out_flat = gk(idx, embedding)
    return out_flat.reshape(b, h, d)


# speedup vs baseline: 3.6697x; 1.0098x over previous
import functools

import jax
import jax.numpy as jnp
from jax import lax
from jax.experimental import pallas as pl
from jax.experimental.pallas import tpu as pltpu
from jax.experimental.pallas import tpu_sc as plsc

_IDX_BLK = 128
_G = 4
_CHUNK = _G * _IDX_BLK
_NBUF = 2


@functools.lru_cache(maxsize=None)
def _build_gather(n_idx: int, n_rows: int, d: int):
    nc, ns = 2, 16
    nw = nc * ns
    rows_per_w = n_idx // (nw * _IDX_BLK)
    assert rows_per_w * nw * _IDX_BLK == n_idx
    steps = rows_per_w // _G
    assert steps * _G == rows_per_w and steps % _NBUF == 0 and steps >= 2 * _NBUF

    mesh = plsc.VectorSubcoreMesh(core_axis_name="c", subcore_axis_name="s")

    @functools.partial(
        pl.kernel,
        mesh=mesh,
        out_type=jax.ShapeDtypeStruct((n_idx, d), jnp.float32),
        scratch_types=[
            pltpu.VMEM((_NBUF * _G, _IDX_BLK), jnp.int32),
            pltpu.VMEM((_NBUF * _CHUNK, d), jnp.float32),
            [pltpu.SemaphoreType.DMA] * _NBUF,
            [pltpu.SemaphoreType.DMA] * _NBUF,
            [pltpu.SemaphoreType.DMA] * _NBUF,
        ],
        compiler_params=pltpu.CompilerParams(use_tc_tiling_on_sc=False),
    )
    def gk(idx_hbm, table_hbm, out_hbm, idx_v, rows_v, sem_i, sem_g, sem_o):
        wid = lax.axis_index("s") * nc + lax.axis_index("c")
        base_row = wid * rows_per_w

        def idx_copy(s, b):
            return pltpu.make_async_copy(
                idx_hbm.at[pl.ds(base_row + s * _G, _G)],
                idx_v.at[pl.ds(b * _G, _G)],
                sem_i[b],
            )

        def out_copy(s, b):
            return pltpu.make_async_copy(
                rows_v.at[pl.ds(b * _CHUNK, _CHUNK)],
                out_hbm.at[pl.ds((base_row + s * _G) * _IDX_BLK, _CHUNK)],
                sem_o[b],
            )

        def gather_copy(b, j):
            return pltpu.make_async_copy(
                table_hbm.at[idx_v.at[b * _G + j]],
                rows_v.at[pl.ds(b * _CHUNK + j * _IDX_BLK, _IDX_BLK)],
                sem_g[b],
            )

        for b in range(_NBUF):
            idx_copy(b, b).start()

        def body(g, carry):
            for b in range(_NBUF):
                s = g + b
                idx_copy(s, b).wait()

                @pl.when(s >= _NBUF)
                def _():
                    out_copy(s - _NBUF, b).wait()

                for j in range(_G):
                    gather_copy(b, j).start()
                for j in range(_G):
                    gather_copy(b, j).wait()

                @pl.when(s + _NBUF < steps)
                def _():
                    idx_copy(s + _NBUF, b).start()

                out_copy(s, b).start()
            return carry

        lax.fori_loop(0, steps // _NBUF, lambda i, c: body(i * _NBUF, c), 0,
                      unroll=False)

        for b in range(_NBUF):
            out_copy(steps - _NBUF + b, b).wait()

    return gk


def kernel(tc, embedding):
    b, h = tc.shape
    v, d = embedding.shape
    n = b * h
    idx = tc.reshape(n // _IDX_BLK, _IDX_BLK).astype(jnp.int32)
    gk = _build_gather(n, v, d)
    out_flat = gk(idx, embedding)
    return out_flat.reshape(b, h, d)
